# baseline (speedup 1.0000x reference)
import jax
import jax.numpy as jnp
from jax import lax
from jax.experimental import pallas as pl
from jax.experimental.pallas import tpu as pltpu
from jax.experimental.pallas import tpu_sc as plsc

_N_SRC = 10000
_N_DST1 = 5000
_N_DST2 = 1000
_D_IN = 128
_HID = 16
_D_OUT = 128

_NC = 2
_NS = 16
_NW = _NC * _NS
_GRP = 128

_E1P = 327680
_E2P = 163840
_G1 = _E1P // (_NW * _GRP)
_G2 = _E2P // (_NW * _GRP)
_ND1P = 5120
_ND2P = 1024


def _leaky(x):
    return jnp.where(x > 0, x, 0.01 * x)


def _proj_body(h_ref, h8_ref, wn_ref, wsbd_ref, p_ref, s_ref):
    p_ref[pl.ds(0, _N_SRC), :] = jnp.dot(h_ref[...], wn_ref[...],
                                         preferred_element_type=jnp.float32)
    s_ref[...] = jnp.dot(h8_ref[...], wsbd_ref[...],
                         preferred_element_type=jnp.float32)


def _out_body(h1_ref, agg_ref, deg_ref, wsbd_ref, wnbd_ref, b_ref, o_ref):
    nf = _N_DST2 // 8
    a = agg_ref[0] + agg_ref[1]
    d = deg_ref[0] + deg_ref[1]
    hn = (a / jnp.maximum(d, 1.0))[:nf]
    x = (jnp.dot(h1_ref[:nf], wsbd_ref[...],
                 preferred_element_type=jnp.float32)
         + jnp.dot(hn, wnbd_ref[...], preferred_element_type=jnp.float32)
         + b_ref[...])
    o_ref[...] = _leaky(x)


def _make_sc_aggregate(n_dst_pad, groups_per_tile, n_src_pad,
                       build_h1=False):
    gpt = groups_per_tile
    zr = n_dst_pad // _NS
    sr = n_src_pad // _NS

    mesh = plsc.VectorSubcoreMesh(core_axis_name="core",
                                  subcore_axis_name="subcore")

    def body(*refs):
        if build_h1:
            (s_hbm, agg_in, deg_in, b_hbm, src_hbm, dst_hbm, ew_hbm,
             agg_out, deg_out, h1_out,
             src_v, dst_v, ew_v, rows0, rows1, rows2, rows3,
             wrows0, wrows1, wrows2, wrows3, ones_v, zrow_v,
             abuf, abuf2, dbuf, dbuf2, hbuf, bbuf,
             p_sp, agg_sp, deg_sp, gsem0, gsem1, gsem2, gsem3,
             ssem0, ssem1, ssem2, ssem3) = refs
        else:
            (p_hbm, src_hbm, dst_hbm, ew_hbm, agg_out, deg_out,
             src_v, dst_v, ew_v, rows0, rows1, rows2, rows3,
             wrows0, wrows1, wrows2, wrows3, ones_v, zrow_v,
             p_sp, agg_sp, deg_sp, gsem0, gsem1, gsem2, gsem3,
             ssem0, ssem1, ssem2, ssem3) = refs
        cid = lax.axis_index("core")
        sid = lax.axis_index("subcore")
        wid = sid * _NC + cid

        slots = ((rows0, wrows0, gsem0, ssem0),
                 (rows1, wrows1, gsem1, ssem1),
                 (rows2, wrows2, gsem2, ssem2),
                 (rows3, wrows3, gsem3, ssem3))

        if build_h1:
            stripe = pl.ds(sid * sr, sr)
            pltpu.sync_copy(b_hbm, bbuf)
            pltpu.sync_copy(s_hbm.at[stripe], hbuf)
            pltpu.sync_copy(agg_in.at[0, stripe], abuf)
            pltpu.sync_copy(agg_in.at[1, stripe], abuf2)
            pltpu.sync_copy(deg_in.at[0, stripe], dbuf)
            pltpu.sync_copy(deg_in.at[1, stripe], dbuf2)

            @pl.loop(0, sr)
            def _(i):
                a = abuf[i, :] + abuf2[i, :]
                d = jnp.maximum(dbuf[i, :] + dbuf2[i, :], 1.0)
                x = hbuf[i, :] + bbuf[...] + a / d
                hbuf[i, :] = jnp.where(x > 0, x, x * 0.01)

            pltpu.sync_copy(hbuf, p_sp.at[stripe])
            pltpu.sync_copy(hbuf, h1_out.at[stripe])
        else:
            pltpu.sync_copy(p_hbm.at[pl.ds(sid * sr, sr)],
                            p_sp.at[pl.ds(sid * sr, sr)])
        pltpu.sync_copy(src_hbm.at[pl.ds(wid * gpt, gpt)], src_v)
        pltpu.sync_copy(dst_hbm.at[pl.ds(wid * gpt, gpt)], dst_v)
        pltpu.sync_copy(ew_hbm.at[pl.ds(wid * gpt * _GRP, gpt * _GRP)],
                        ew_v)

        @pl.loop(0, _GRP)
        def _(i):
            ones_v[i, :] = jnp.ones((16,), jnp.float32)

        @pl.loop(0, zr)
        def _(i):
            zrow_v[i, :] = jnp.zeros((16,), jnp.float32)

        pltpu.sync_copy(zrow_v, agg_sp.at[pl.ds(sid * zr, zr)])
        pltpu.sync_copy(zrow_v, deg_sp.at[pl.ds(sid * zr, zr)])
        plsc.subcore_barrier()

        for b, (rows, _, gsem, _) in enumerate(slots):
            pltpu.async_copy(p_sp.at[src_v.at[b]], rows, gsem)

        def step(gb, rows, wrows, gsem, ssem, first):
            pltpu.make_async_copy(p_sp.at[src_v.at[gb]], rows,
                                  gsem).wait()
            if not first:
                pltpu.make_async_copy(wrows, agg_sp.at[dst_v.at[gb]],
                                      ssem).wait()
                pltpu.make_async_copy(ones_v, deg_sp.at[dst_v.at[gb]],
                                      ssem).wait()

            @pl.loop(0, _GRP // 16)
            def _(j):
                w16 = ew_v[pl.ds(gb * _GRP + j * 16, 16)]
                base = j * 16
                for r in range(16):
                    wv = jnp.take_along_axis(
                        w16, jnp.full((16,), r, jnp.int32), axis=0)
                    wrows[base + r, :] = rows[base + r, :] * wv

            pltpu.async_copy(wrows, agg_sp.at[dst_v.at[gb]], ssem,
                             add=True)
            pltpu.async_copy(ones_v, deg_sp.at[dst_v.at[gb]], ssem,
                             add=True)
            gnext = lax.rem(gb + 4, gpt)
            pltpu.async_copy(p_sp.at[src_v.at[gnext]], rows, gsem)

        for b, (rows, wrows, gsem, ssem) in enumerate(slots):
            step(b, rows, wrows, gsem, ssem, first=True)

        @pl.loop(4, gpt, step=4)
        def _(g):
            for b, (rows, wrows, gsem, ssem) in enumerate(slots):
                step(g + b, rows, wrows, gsem, ssem, first=False)

        for b, (rows, wrows, gsem, ssem) in enumerate(slots):
            pltpu.make_async_copy(p_sp.at[src_v.at[b]], rows, gsem).wait()
            pltpu.make_async_copy(wrows, agg_sp.at[dst_v.at[b]],
                                  ssem).wait()
            pltpu.make_async_copy(ones_v, deg_sp.at[dst_v.at[b]],
                                  ssem).wait()

        plsc.subcore_barrier()

        pltpu.sync_copy(agg_sp.at[pl.ds(sid * zr, zr)],
                        agg_out.at[cid, pl.ds(sid * zr, zr)])
        pltpu.sync_copy(deg_sp.at[pl.ds(sid * zr, zr)],
                        deg_out.at[cid, pl.ds(sid * zr, zr)])

    out_type = [jax.ShapeDtypeStruct((_NC, n_dst_pad, 16), jnp.float32),
                jax.ShapeDtypeStruct((_NC, n_dst_pad, 16), jnp.float32)]
    if build_h1:
        out_type.append(
            jax.ShapeDtypeStruct((n_src_pad, 16), jnp.float32))
    scratch = (
        [pltpu.VMEM((gpt, _GRP), jnp.int32),
         pltpu.VMEM((gpt, _GRP), jnp.int32),
         pltpu.VMEM((gpt * _GRP,), jnp.float32)]
        + [pltpu.VMEM((_GRP, 16), jnp.float32)] * 8
        + [pltpu.VMEM((_GRP, 16), jnp.float32),
           pltpu.VMEM((zr, 16), jnp.float32)]
        + ([pltpu.VMEM((sr, 16), jnp.float32)] * 5
           + [pltpu.VMEM((16,), jnp.float32)]
           if build_h1 else [])
        + [pltpu.VMEM_SHARED((n_src_pad, 16), jnp.float32),
           pltpu.VMEM_SHARED((n_dst_pad, 16), jnp.float32),
           pltpu.VMEM_SHARED((n_dst_pad, 16), jnp.float32)]
        + [pltpu.SemaphoreType.DMA] * 8
    )
    cp = pltpu.CompilerParams(needs_layout_passes=False,
                              use_tc_tiling_on_sc=False)
    return pl.kernel(body, out_type=out_type, mesh=mesh,
                     scratch_types=scratch, compiler_params=cp)


_N_SRCP = 10240
_sc_agg1 = _make_sc_aggregate(_ND1P, _G1, _N_SRCP)
_sc_agg2 = _make_sc_aggregate(_ND2P, _G2, _ND1P, build_h1=True)


def _pad_edges(src, dst, ew, e_pad, n_dst):
    e = src.shape[0]
    src = jnp.pad(src, (0, e_pad - e)).reshape(-1, _GRP)
    dst = jnp.pad(dst, (0, e_pad - e),
                  constant_values=n_dst).reshape(-1, _GRP)
    ew = jnp.pad(ew, (0, e_pad - e))
    return src, dst, ew


def _block_diag8(w):
    k, m = w.shape
    out = jnp.zeros((8, k, 8, m), w.dtype)
    idx = jnp.arange(8)
    out = out.at[idx, :, idx, :].set(w)
    return out.reshape(8 * k, 8 * m)


def kernel(node_feat, edge_index1, edge_index2, edge_weight1, edge_weight2,
           W_self1, W_neigh1, b1, W_self2, W_neigh2, b2):
    h = node_feat.reshape(_N_SRC, _D_IN)

    src1, dst1, ew1 = _pad_edges(edge_index1[0], edge_index1[1],
                                 edge_weight1, _E1P, _N_DST1)
    src2, dst2, ew2 = _pad_edges(edge_index2[0], edge_index2[1],
                                 edge_weight2, _E2P, _N_DST2)

    h8 = h[:_ND1P].reshape(_ND1P // 8, 8 * _D_IN)
    p1, s1f = pl.pallas_call(
        _proj_body,
        out_shape=[
            jax.ShapeDtypeStruct((_N_SRCP, _HID), jnp.float32),
            jax.ShapeDtypeStruct((_ND1P // 8, 128), jnp.float32),
        ],
    )(h, h8, W_neigh1, _block_diag8(W_self1))

    agg1, deg1 = _sc_agg1(p1, src1, dst1, ew1)

    agg2, deg2, h1 = _sc_agg2(s1f.reshape(_ND1P, _HID), agg1, deg1, b1,
                              src2, dst2, ew2)
    h1f = h1.reshape(_ND1P // 8, 128)

    outf = pl.pallas_call(
        _out_body,
        out_shape=jax.ShapeDtypeStruct((_N_DST2 // 8, 8 * _D_OUT),
                                       jnp.float32),
    )(h1f, agg2.reshape(_NC, _ND2P // 8, 128),
      deg2.reshape(_NC, _ND2P // 8, 128), _block_diag8(W_self2),
      _block_diag8(W_neigh2), jnp.tile(b2, 8).reshape(1, 8 * _D_OUT))

    return outf.reshape(1, _N_DST2, _D_OUT)


# speedup vs baseline: 1.0523x; 1.0523x over previous
import jax
import jax.numpy as jnp
from jax import lax
from jax.experimental import pallas as pl
from jax.experimental.pallas import tpu as pltpu
from jax.experimental.pallas import tpu_sc as plsc

_N_SRC = 10000
_N_DST1 = 5000
_N_DST2 = 1000
_D_IN = 128
_HID = 16
_D_OUT = 128

_NC = 2
_NS = 16
_NW = _NC * _NS
_GRP = 128

_E1P = 327680
_E2P = 163840
_G1 = _E1P // (_NW * _GRP)
_G2 = _E2P // (_NW * _GRP)
_ND1P = 5120
_ND2P = 1024


def _leaky(x):
    return jnp.where(x > 0, x, 0.01 * x)


def _proj_body(h_ref, h8_ref, wn_ref, wsbd_ref, p_ref, s_ref):
    p_ref[pl.ds(0, _N_SRC), :] = jnp.dot(h_ref[...], wn_ref[...],
                                         preferred_element_type=jnp.float32)
    s_ref[...] = jnp.dot(h8_ref[...], wsbd_ref[...],
                         preferred_element_type=jnp.float32)


def _h1_body(s_ref, agg_ref, deg_ref, b_ref, h1_ref):
    a = agg_ref[0] + agg_ref[1]
    d = deg_ref[0] + deg_ref[1]
    x = s_ref[...] + a / jnp.maximum(d, 1.0) + b_ref[...]
    h1_ref[...] = _leaky(x)


def _out_body(h1_ref, agg_ref, deg_ref, wsbd_ref, wnbd_ref, b_ref, o_ref):
    nf = _N_DST2 // 8
    a = agg_ref[0] + agg_ref[1]
    d = deg_ref[0] + deg_ref[1]
    hn = (a / jnp.maximum(d, 1.0))[:nf]
    x = (jnp.dot(h1_ref[:nf], wsbd_ref[...],
                 preferred_element_type=jnp.float32)
         + jnp.dot(hn, wnbd_ref[...], preferred_element_type=jnp.float32)
         + b_ref[...])
    o_ref[...] = _leaky(x)


def _make_sc_aggregate(n_dst_pad, groups_per_tile, n_src_pad):
    gpt = groups_per_tile
    zr = n_dst_pad // _NS
    sr = n_src_pad // _NS

    mesh = plsc.VectorSubcoreMesh(core_axis_name="core",
                                  subcore_axis_name="subcore")

    def body(p_hbm, src_hbm, dst_hbm, ew_hbm, agg_out, deg_out,
             src_v, dst_v, ew_v, rows0, rows1, rows2, rows3,
             wrows0, wrows1, wrows2, wrows3, ones_v, zrow_v,
             p_sp, agg_sp, deg_sp, gsem0, gsem1, gsem2, gsem3,
             ssem0, ssem1, ssem2, ssem3):
        cid = lax.axis_index("core")
        sid = lax.axis_index("subcore")
        wid = sid * _NC + cid

        slots = ((rows0, wrows0, gsem0, ssem0),
                 (rows1, wrows1, gsem1, ssem1),
                 (rows2, wrows2, gsem2, ssem2),
                 (rows3, wrows3, gsem3, ssem3))

        pltpu.sync_copy(p_hbm.at[pl.ds(sid * sr, sr)],
                        p_sp.at[pl.ds(sid * sr, sr)])
        pltpu.sync_copy(src_hbm.at[pl.ds(wid * gpt, gpt)], src_v)
        pltpu.sync_copy(dst_hbm.at[pl.ds(wid * gpt, gpt)], dst_v)
        pltpu.sync_copy(ew_hbm.at[pl.ds(wid * gpt * _GRP, gpt * _GRP)],
                        ew_v)

        @pl.loop(0, _GRP)
        def _(i):
            ones_v[i, :] = jnp.ones((16,), jnp.float32)

        @pl.loop(0, zr)
        def _(i):
            zrow_v[i, :] = jnp.zeros((16,), jnp.float32)

        pltpu.sync_copy(zrow_v, agg_sp.at[pl.ds(sid * zr, zr)])
        pltpu.sync_copy(zrow_v, deg_sp.at[pl.ds(sid * zr, zr)])
        plsc.subcore_barrier()

        for b, (rows, _, gsem, _) in enumerate(slots):
            pltpu.async_copy(p_sp.at[src_v.at[b]], rows, gsem)

        def step(gb, rows, wrows, gsem, ssem, first):
            pltpu.make_async_copy(p_sp.at[src_v.at[gb]], rows,
                                  gsem).wait()
            if not first:
                pltpu.make_async_copy(wrows, agg_sp.at[dst_v.at[gb]],
                                      ssem).wait()
                pltpu.make_async_copy(ones_v, deg_sp.at[dst_v.at[gb]],
                                      ssem).wait()

            @pl.loop(0, _GRP // 16)
            def _(j):
                w16 = ew_v[pl.ds(gb * _GRP + j * 16, 16)]
                base = j * 16
                for r in range(16):
                    wv = jnp.take_along_axis(
                        w16, jnp.full((16,), r, jnp.int32), axis=0)
                    wrows[base + r, :] = rows[base + r, :] * wv

            pltpu.async_copy(wrows, agg_sp.at[dst_v.at[gb]], ssem,
                             add=True)
            pltpu.async_copy(ones_v, deg_sp.at[dst_v.at[gb]], ssem,
                             add=True)
            gnext = lax.rem(gb + 4, gpt)
            pltpu.async_copy(p_sp.at[src_v.at[gnext]], rows, gsem)

        for b, (rows, wrows, gsem, ssem) in enumerate(slots):
            step(b, rows, wrows, gsem, ssem, first=True)

        @pl.loop(4, gpt, step=4)
        def _(g):
            for b, (rows, wrows, gsem, ssem) in enumerate(slots):
                step(g + b, rows, wrows, gsem, ssem, first=False)

        for b, (rows, wrows, gsem, ssem) in enumerate(slots):
            pltpu.make_async_copy(p_sp.at[src_v.at[b]], rows, gsem).wait()
            pltpu.make_async_copy(wrows, agg_sp.at[dst_v.at[b]],
                                  ssem).wait()
            pltpu.make_async_copy(ones_v, deg_sp.at[dst_v.at[b]],
                                  ssem).wait()

        plsc.subcore_barrier()

        pltpu.sync_copy(agg_sp.at[pl.ds(sid * zr, zr)],
                        agg_out.at[cid, pl.ds(sid * zr, zr)])
        pltpu.sync_copy(deg_sp.at[pl.ds(sid * zr, zr)],
                        deg_out.at[cid, pl.ds(sid * zr, zr)])

    out_type = [jax.ShapeDtypeStruct((_NC, n_dst_pad, 16), jnp.float32),
                jax.ShapeDtypeStruct((_NC, n_dst_pad, 16), jnp.float32)]
    scratch = (
        [pltpu.VMEM((gpt, _GRP), jnp.int32),
         pltpu.VMEM((gpt, _GRP), jnp.int32),
         pltpu.VMEM((gpt * _GRP,), jnp.float32)]
        + [pltpu.VMEM((_GRP, 16), jnp.float32)] * 8
        + [pltpu.VMEM((_GRP, 16), jnp.float32),
           pltpu.VMEM((zr, 16), jnp.float32)]
        + [pltpu.VMEM_SHARED((n_src_pad, 16), jnp.float32),
           pltpu.VMEM_SHARED((n_dst_pad, 16), jnp.float32),
           pltpu.VMEM_SHARED((n_dst_pad, 16), jnp.float32)]
        + [pltpu.SemaphoreType.DMA] * 8
    )
    cp = pltpu.CompilerParams(needs_layout_passes=False,
                              use_tc_tiling_on_sc=False)
    return pl.kernel(body, out_type=out_type, mesh=mesh,
                     scratch_types=scratch, compiler_params=cp)


_N_SRCP = 10240
_sc_agg1 = _make_sc_aggregate(_ND1P, _G1, _N_SRCP)
_sc_agg2 = _make_sc_aggregate(_ND2P, _G2, _ND1P)


def _pad_edges(src, dst, ew, e_pad, n_dst):
    e = src.shape[0]
    src = jnp.pad(src, (0, e_pad - e)).reshape(-1, _GRP)
    dst = jnp.pad(dst, (0, e_pad - e),
                  constant_values=n_dst).reshape(-1, _GRP)
    ew = jnp.pad(ew, (0, e_pad - e))
    return src, dst, ew


def _block_diag8(w):
    k, m = w.shape
    out = jnp.zeros((8, k, 8, m), w.dtype)
    idx = jnp.arange(8)
    out = out.at[idx, :, idx, :].set(w)
    return out.reshape(8 * k, 8 * m)


def kernel(node_feat, edge_index1, edge_index2, edge_weight1, edge_weight2,
           W_self1, W_neigh1, b1, W_self2, W_neigh2, b2):
    h = node_feat.reshape(_N_SRC, _D_IN)

    src1, dst1, ew1 = _pad_edges(edge_index1[0], edge_index1[1],
                                 edge_weight1, _E1P, _N_DST1)
    src2, dst2, ew2 = _pad_edges(edge_index2[0], edge_index2[1],
                                 edge_weight2, _E2P, _N_DST2)

    h8 = h[:_ND1P].reshape(_ND1P // 8, 8 * _D_IN)
    p1, s1f = pl.pallas_call(
        _proj_body,
        out_shape=[
            jax.ShapeDtypeStruct((_N_SRCP, _HID), jnp.float32),
            jax.ShapeDtypeStruct((_ND1P // 8, 128), jnp.float32),
        ],
    )(h, h8, W_neigh1, _block_diag8(W_self1))

    agg1, deg1 = _sc_agg1(p1, src1, dst1, ew1)

    h1f = pl.pallas_call(
        _h1_body,
        out_shape=jax.ShapeDtypeStruct((_ND1P // 8, 128), jnp.float32),
    )(s1f, agg1.reshape(_NC, _ND1P // 8, 128),
      deg1.reshape(_NC, _ND1P // 8, 128), jnp.tile(b1, 8).reshape(1, 128))

    h1 = h1f.reshape(_ND1P, _HID)
    agg2, deg2 = _sc_agg2(h1, src2, dst2, ew2)

    outf = pl.pallas_call(
        _out_body,
        out_shape=jax.ShapeDtypeStruct((_N_DST2 // 8, 8 * _D_OUT),
                                       jnp.float32),
    )(h1f, agg2.reshape(_NC, _ND2P // 8, 128),
      deg2.reshape(_NC, _ND2P // 8, 128), _block_diag8(W_self2),
      _block_diag8(W_neigh2), jnp.tile(b2, 8).reshape(1, 8 * _D_OUT))

    return outf.reshape(1, _N_DST2, _D_OUT)
